# baseline (device time: 75259 ns/iter reference)
import jax
import jax.numpy as jnp
from jax import lax
from jax.experimental import pallas as pl
from jax.experimental.pallas import tpu as pltpu

N_DEV = 16


def kernel(x, w_mat):
    m, k = x.shape
    k2, n = w_mat.shape
    nb = n // N_DEV

    def body(me_ref, x_ref, w_ref, out_ref,
             send_buf, recv_buf, send_sems, recv_sems):
        t = pl.program_id(0)
        me = lax.axis_index("i")
        j = lax.rem(me + t, N_DEV)

        xb = x_ref[...].astype(jnp.bfloat16)
        wb = w_ref[...].astype(jnp.bfloat16)
        block = jnp.dot(xb, wb, preferred_element_type=jnp.float32)
        block = block.astype(jnp.bfloat16)
        send_buf[pl.ds(j, 1)] = block[None]

        @pl.when(t == 0)
        def _():
            recv_buf[pl.ds(me, 1)] = block[None]

        @pl.when(t != 0)
        def _():
            rdma = pltpu.make_async_remote_copy(
                src_ref=send_buf.at[j],
                dst_ref=recv_buf.at[me],
                send_sem=send_sems.at[j],
                recv_sem=recv_sems.at[me],
                device_id=(j,),
                device_id_type=pl.DeviceIdType.MESH,
            )
            rdma.start()

        @pl.when(t == N_DEV - 1)
        def _():
            for kk in range(N_DEV):
                @pl.when(kk != me)
                def _(kk=kk):
                    pltpu.make_async_remote_copy(
                        src_ref=send_buf.at[kk],
                        dst_ref=recv_buf.at[me],
                        send_sem=send_sems.at[kk],
                        recv_sem=recv_sems.at[me],
                        device_id=(me,),
                        device_id_type=pl.DeviceIdType.MESH,
                    ).wait_send()
                    pltpu.make_async_remote_copy(
                        src_ref=send_buf.at[kk],
                        dst_ref=recv_buf.at[kk],
                        send_sem=send_sems.at[kk],
                        recv_sem=recv_sems.at[kk],
                        device_id=(me,),
                        device_id_type=pl.DeviceIdType.MESH,
                    ).wait_recv()
                out_ref[pl.ds(kk * m, m), :] = recv_buf[kk].astype(jnp.float32)

    me = lax.axis_index("i").astype(jnp.int32)
    me_arr = jnp.reshape(me, (1,))

    grid_spec = pltpu.PrefetchScalarGridSpec(
        num_scalar_prefetch=1,
        grid=(N_DEV,),
        in_specs=[
            pl.BlockSpec((m, k), lambda t, me_r: (0, 0)),
            pl.BlockSpec(
                (k2, nb), lambda t, me_r: (0, (me_r[0] + t) % N_DEV)
            ),
        ],
        out_specs=pl.BlockSpec((N_DEV * m, nb), lambda t, me_r: (0, 0)),
        scratch_shapes=[
            pltpu.VMEM((N_DEV, m, nb), jnp.bfloat16),
            pltpu.VMEM((N_DEV, m, nb), jnp.bfloat16),
            pltpu.SemaphoreType.DMA((N_DEV,)),
            pltpu.SemaphoreType.DMA((N_DEV,)),
        ],
    )

    return pl.pallas_call(
        body,
        grid_spec=grid_spec,
        out_shape=jax.ShapeDtypeStruct((N_DEV * m, nb), jnp.float32),
        compiler_params=pltpu.CompilerParams(
            dimension_semantics=("arbitrary",),
        ),
    )(me_arr, x, w_mat)


# device time: 71215 ns/iter; 1.0568x vs baseline; 1.0568x over previous
import jax
import jax.numpy as jnp
from jax import lax
from jax.experimental import pallas as pl
from jax.experimental.pallas import tpu as pltpu

N_DEV = 16

SHIFTS = (0, 8, 7, 9, 6, 10, 5, 11, 4, 12, 3, 13, 2, 14, 1, 15)


def kernel(x, w_mat):
    m, k = x.shape
    k2, n = w_mat.shape
    nb = n // N_DEV

    def body(me_ref, shifts_ref, x_ref, w_ref, out_ref,
             send_buf, recv_buf, send_sems, recv_sems):
        t = pl.program_id(0)
        me = lax.axis_index("i")
        j = lax.rem(me + shifts_ref[t], N_DEV)

        xb = x_ref[...].astype(jnp.bfloat16)
        wb = w_ref[...].astype(jnp.bfloat16)
        block = jnp.dot(xb, wb, preferred_element_type=jnp.float32)
        block = block.astype(jnp.bfloat16)
        send_buf[pl.ds(j, 1)] = block[None]

        @pl.when(t == 0)
        def _():
            recv_buf[pl.ds(me, 1)] = block[None]

        @pl.when(t != 0)
        def _():
            rdma = pltpu.make_async_remote_copy(
                src_ref=send_buf.at[j],
                dst_ref=recv_buf.at[me],
                send_sem=send_sems.at[j],
                recv_sem=recv_sems.at[me],
                device_id=(j,),
                device_id_type=pl.DeviceIdType.MESH,
            )
            rdma.start()

        @pl.when(t == N_DEV - 1)
        def _():
            for s in SHIFTS:
                if s == 0:
                    out_ref[pl.ds(me * m, m), :] = (
                        recv_buf[me].astype(jnp.float32))
                    continue
                kk = lax.rem(me - s + N_DEV, N_DEV)
                jj = lax.rem(me + s, N_DEV)
                pltpu.make_async_remote_copy(
                    src_ref=send_buf.at[jj],
                    dst_ref=recv_buf.at[me],
                    send_sem=send_sems.at[jj],
                    recv_sem=recv_sems.at[me],
                    device_id=(me,),
                    device_id_type=pl.DeviceIdType.MESH,
                ).wait_send()
                pltpu.make_async_remote_copy(
                    src_ref=send_buf.at[jj],
                    dst_ref=recv_buf.at[kk],
                    send_sem=send_sems.at[jj],
                    recv_sem=recv_sems.at[kk],
                    device_id=(me,),
                    device_id_type=pl.DeviceIdType.MESH,
                ).wait_recv()
                out_ref[pl.ds(kk * m, m), :] = recv_buf[kk].astype(jnp.float32)

    me = lax.axis_index("i").astype(jnp.int32)
    me_arr = jnp.reshape(me, (1,))
    shifts_arr = jnp.asarray(SHIFTS, dtype=jnp.int32)

    grid_spec = pltpu.PrefetchScalarGridSpec(
        num_scalar_prefetch=2,
        grid=(N_DEV,),
        in_specs=[
            pl.BlockSpec((m, k), lambda t, me_r, s_r: (0, 0)),
            pl.BlockSpec(
                (k2, nb),
                lambda t, me_r, s_r: (0, (me_r[0] + s_r[t]) % N_DEV),
            ),
        ],
        out_specs=pl.BlockSpec((N_DEV * m, nb), lambda t, me_r, s_r: (0, 0)),
        scratch_shapes=[
            pltpu.VMEM((N_DEV, m, nb), jnp.bfloat16),
            pltpu.VMEM((N_DEV, m, nb), jnp.bfloat16),
            pltpu.SemaphoreType.DMA((N_DEV,)),
            pltpu.SemaphoreType.DMA((N_DEV,)),
        ],
    )

    return pl.pallas_call(
        body,
        grid_spec=grid_spec,
        out_shape=jax.ShapeDtypeStruct((N_DEV * m, nb), jnp.float32),
        compiler_params=pltpu.CompilerParams(
            dimension_semantics=("arbitrary",),
        ),
    )(me_arr, shifts_arr, x, w_mat)


# device time: 70035 ns/iter; 1.0746x vs baseline; 1.0168x over previous
import jax
import jax.numpy as jnp
from jax import lax
from jax.experimental import pallas as pl
from jax.experimental.pallas import tpu as pltpu

N_DEV = 16

SHIFTS = (0, 8, 7, 9, 6, 10, 5, 11, 4, 12, 3, 13, 2, 14, 1, 15)


def kernel(x, w_mat):
    m, k = x.shape
    k2, n = w_mat.shape
    nb = n // N_DEV

    def body(me_ref, shifts_ref, x_ref, w_ref, out_ref,
             send_buf, send_sems, recv_sems):
        t = pl.program_id(0)
        me = lax.axis_index("i")
        j = lax.rem(me + shifts_ref[t], N_DEV)

        xb = x_ref[...].astype(jnp.bfloat16)
        wb = w_ref[...].astype(jnp.bfloat16)
        block = jnp.dot(xb, wb, preferred_element_type=jnp.float32)
        block = block.astype(jnp.bfloat16)

        @pl.when(t == 0)
        def _():
            out_ref[pl.ds(me * m, m), :] = block

        @pl.when(t != 0)
        def _():
            send_buf[pl.ds(j, 1)] = block[None]
            rdma = pltpu.make_async_remote_copy(
                src_ref=send_buf.at[j],
                dst_ref=out_ref.at[pl.ds(me * m, m), :],
                send_sem=send_sems.at[j],
                recv_sem=recv_sems.at[me],
                device_id=(j,),
                device_id_type=pl.DeviceIdType.MESH,
            )
            rdma.start()

        @pl.when(t == N_DEV - 1)
        def _():
            for s in SHIFTS:
                if s == 0:
                    continue
                kk = lax.rem(me - s + N_DEV, N_DEV)
                jj = lax.rem(me + s, N_DEV)
                pltpu.make_async_remote_copy(
                    src_ref=send_buf.at[jj],
                    dst_ref=out_ref.at[pl.ds(me * m, m), :],
                    send_sem=send_sems.at[jj],
                    recv_sem=recv_sems.at[me],
                    device_id=(me,),
                    device_id_type=pl.DeviceIdType.MESH,
                ).wait_send()
                pltpu.make_async_remote_copy(
                    src_ref=send_buf.at[jj],
                    dst_ref=out_ref.at[pl.ds(kk * m, m), :],
                    send_sem=send_sems.at[jj],
                    recv_sem=recv_sems.at[kk],
                    device_id=(me,),
                    device_id_type=pl.DeviceIdType.MESH,
                ).wait_recv()

    me = lax.axis_index("i").astype(jnp.int32)
    me_arr = jnp.reshape(me, (1,))
    shifts_arr = jnp.asarray(SHIFTS, dtype=jnp.int32)

    grid_spec = pltpu.PrefetchScalarGridSpec(
        num_scalar_prefetch=2,
        grid=(N_DEV,),
        in_specs=[
            pl.BlockSpec((m, k), lambda t, me_r, s_r: (0, 0)),
            pl.BlockSpec(
                (k2, nb),
                lambda t, me_r, s_r: (0, (me_r[0] + s_r[t]) % N_DEV),
            ),
        ],
        out_specs=pl.BlockSpec((N_DEV * m, nb), lambda t, me_r, s_r: (0, 0)),
        scratch_shapes=[
            pltpu.VMEM((N_DEV, m, nb), jnp.bfloat16),
            pltpu.SemaphoreType.DMA((N_DEV,)),
            pltpu.SemaphoreType.DMA((N_DEV,)),
        ],
    )

    return pl.pallas_call(
        body,
        grid_spec=grid_spec,
        out_shape=jax.ShapeDtypeStruct((N_DEV * m, nb), jnp.bfloat16),
        compiler_params=pltpu.CompilerParams(
            dimension_semantics=("arbitrary",),
        ),
    )(me_arr, shifts_arr, x, w_mat)


# device time: 69920 ns/iter; 1.0764x vs baseline; 1.0016x over previous
import jax
import jax.numpy as jnp
from jax import lax
from jax.experimental import pallas as pl
from jax.experimental.pallas import tpu as pltpu

N_DEV = 16

SHIFTS = (0, 8, 7, 9, 6, 10, 5, 11, 4, 12, 3, 13, 2, 14, 1, 15)


def kernel(x, w_mat):
    m, k = x.shape
    k2, n = w_mat.shape
    nb = n // N_DEV

    def body(me_ref, shifts_ref, x_ref, w_ref, out_ref,
             send_buf, send_sems, recv_sems):
        t = pl.program_id(0)
        me = lax.axis_index("i")
        j = lax.rem(me + shifts_ref[t], N_DEV)

        xb = x_ref[...].astype(jnp.bfloat16)
        wb = w_ref[...].astype(jnp.bfloat16)
        block = jnp.dot(xb, wb, preferred_element_type=jnp.float32)
        send_buf[pl.ds(j, 1)] = block.astype(jnp.bfloat16)[None]

        @pl.when(t == 0)
        def _():
            pltpu.make_async_copy(
                send_buf.at[me],
                out_ref.at[pl.ds(me * m, m), :],
                send_sems.at[me],
            ).start()

        @pl.when(t != 0)
        def _():
            rdma = pltpu.make_async_remote_copy(
                src_ref=send_buf.at[j],
                dst_ref=out_ref.at[pl.ds(me * m, m), :],
                send_sem=send_sems.at[j],
                recv_sem=recv_sems.at[me],
                device_id=(j,),
                device_id_type=pl.DeviceIdType.MESH,
            )
            rdma.start()

        @pl.when(t == N_DEV - 1)
        def _():
            pltpu.make_async_copy(
                send_buf.at[me],
                out_ref.at[pl.ds(me * m, m), :],
                send_sems.at[me],
            ).wait()
            for s in SHIFTS:
                if s == 0:
                    continue
                kk = lax.rem(me - s + N_DEV, N_DEV)
                jj = lax.rem(me + s, N_DEV)
                pltpu.make_async_remote_copy(
                    src_ref=send_buf.at[jj],
                    dst_ref=out_ref.at[pl.ds(me * m, m), :],
                    send_sem=send_sems.at[jj],
                    recv_sem=recv_sems.at[me],
                    device_id=(me,),
                    device_id_type=pl.DeviceIdType.MESH,
                ).wait_send()
                pltpu.make_async_remote_copy(
                    src_ref=send_buf.at[jj],
                    dst_ref=out_ref.at[pl.ds(kk * m, m), :],
                    send_sem=send_sems.at[jj],
                    recv_sem=recv_sems.at[kk],
                    device_id=(me,),
                    device_id_type=pl.DeviceIdType.MESH,
                ).wait_recv()

    me = lax.axis_index("i").astype(jnp.int32)
    me_arr = jnp.reshape(me, (1,))
    shifts_arr = jnp.asarray(SHIFTS, dtype=jnp.int32)

    grid_spec = pltpu.PrefetchScalarGridSpec(
        num_scalar_prefetch=2,
        grid=(N_DEV,),
        in_specs=[
            pl.BlockSpec((m, k), lambda t, me_r, s_r: (0, 0)),
            pl.BlockSpec(
                (k2, nb),
                lambda t, me_r, s_r: (0, (me_r[0] + s_r[t]) % N_DEV),
            ),
        ],
        out_specs=pl.BlockSpec(memory_space=pl.ANY),
        scratch_shapes=[
            pltpu.VMEM((N_DEV, m, nb), jnp.bfloat16),
            pltpu.SemaphoreType.DMA((N_DEV,)),
            pltpu.SemaphoreType.DMA((N_DEV,)),
        ],
    )

    return pl.pallas_call(
        body,
        grid_spec=grid_spec,
        out_shape=jax.ShapeDtypeStruct((N_DEV * m, nb), jnp.bfloat16),
        compiler_params=pltpu.CompilerParams(
            dimension_semantics=("arbitrary",),
        ),
    )(me_arr, shifts_arr, x, w_mat)
